# baseline (speedup 1.0000x reference)
import functools

import jax
import jax.numpy as jnp
from jax import lax
from jax.experimental import pallas as pl
from jax.experimental.pallas import tpu as pltpu
from jax.experimental.pallas import tpu_sc as plsc

_H = 128
_L = 16
_ROWS_PER_STREAM = 128
_STREAMS_PER_SUPER = 5


def _transpose_block(in_v, out_v, e_rows, n_pairs):
    iota = lax.iota(jnp.int32, _L)

    def row(k, carry):
        for j in range(8):
            w = j * _L
            if w + _L <= e_rows:
                e_vec = w + iota
                col = 2 * k
            elif w >= e_rows:
                e_vec = (w - e_rows) + iota
                col = 2 * k + 1
            else:
                raise AssertionError("e_rows must be a multiple of 16")
            vals = plsc.load_gather(
                in_v, [e_vec, jnp.full((_L,), col, jnp.int32)])
            out_v[k, pl.ds(w, _L)] = vals
        return carry

    lax.fori_loop(0, n_pairs, row, 0)


def _sc_transpose_pack(embT, tail_pairs):
    e_rows, v = embT.shape
    d = 2 * e_rows
    n_full = v // d
    tail_rows = (v - n_full * d) // 2
    info = plsc.get_sparse_core_info()
    nc, ns = info.num_cores, info.num_subcores
    nw = nc * ns
    mesh = plsc.VectorSubcoreMesh(core_axis_name="c", subcore_axis_name="s")

    @functools.partial(
        pl.kernel,
        mesh=mesh,
        out_type=jax.ShapeDtypeStruct((v // 2, d), jnp.float32),
        scratch_types=[
            pltpu.VMEM((e_rows, d), jnp.float32),
            pltpu.VMEM((d // 2, d), jnp.float32),
        ],
        compiler_params=pltpu.CompilerParams(needs_layout_passes=False),
    )
    def k(embT_hbm, tail_hbm, pairs_hbm, in_v, out_v):
        wid = lax.axis_index("s") * nc + lax.axis_index("c")
        n_w = (n_full - wid + nw - 1) // nw

        def blk(s, carry):
            b = wid + s * nw
            c0 = b * d
            pltpu.sync_copy(embT_hbm.at[:, pl.ds(c0, d)], in_v)
            _transpose_block(in_v, out_v, e_rows, d // 2)
            pltpu.sync_copy(out_v, pairs_hbm.at[pl.ds(b * (d // 2), d // 2)])
            return carry

        lax.fori_loop(0, n_w, blk, 0)

        if tail_rows:
            @pl.when(wid == 1)
            def _():
                pltpu.sync_copy(tail_hbm, out_v.at[pl.ds(0, tail_rows)])
                pltpu.sync_copy(
                    out_v.at[pl.ds(0, tail_rows)],
                    pairs_hbm.at[pl.ds(v // 2 - tail_rows, tail_rows)])

    return k(embT, tail_pairs)


def _sc_gather(table, idx2d):
    n_chunks = idx2d.shape[0]
    d = table.shape[1]
    info = plsc.get_sparse_core_info()
    nc, ns = info.num_cores, info.num_subcores
    nw = nc * ns
    ch_per_w = n_chunks // nw
    n_super = ch_per_w // _STREAMS_PER_SUPER
    assert n_chunks % nw == 0 and ch_per_w % _STREAMS_PER_SUPER == 0

    mesh = plsc.VectorSubcoreMesh(core_axis_name="c", subcore_axis_name="s")
    rows_per_super = _STREAMS_PER_SUPER * _ROWS_PER_STREAM

    idx4d = idx2d.reshape(nw, n_super, _STREAMS_PER_SUPER, _ROWS_PER_STREAM)

    @functools.partial(
        pl.kernel,
        mesh=mesh,
        out_type=jax.ShapeDtypeStruct((n_chunks * _ROWS_PER_STREAM, d),
                                      jnp.float32),
        scratch_types=[
            pltpu.VMEM((_STREAMS_PER_SUPER, _ROWS_PER_STREAM), jnp.int32),
            pltpu.VMEM((rows_per_super, d), jnp.float32),
            pltpu.SemaphoreType.DMA,
        ],
    )
    def k(table_hbm, idx_hbm, out_hbm, idx_v, rows_v, sem):
        wid = lax.axis_index("s") * nc + lax.axis_index("c")
        base_ch = wid * ch_per_w

        def body(s, carry):
            ch0 = base_ch + s * _STREAMS_PER_SUPER
            pltpu.sync_copy(idx_hbm.at[wid, s], idx_v)
            copies = [
                pltpu.async_copy(
                    table_hbm.at[idx_v.at[j]],
                    rows_v.at[pl.ds(j * _ROWS_PER_STREAM, _ROWS_PER_STREAM)],
                    sem,
                )
                for j in range(_STREAMS_PER_SUPER)
            ]
            for cp in copies:
                cp.wait()
            pltpu.sync_copy(
                rows_v,
                out_hbm.at[pl.ds(ch0 * _ROWS_PER_STREAM, rows_per_super)],
            )
            return carry

        lax.fori_loop(0, n_super, body, 0)

    return k(table, idx4d)


def _lstm_body(x_ref, p_ref, w2_ref, u_ref, b_ref, wd_ref, bd_ref, o_ref,
               h_ref, c_ref, *, tt, batch, d):
    ti = pl.program_id(0)

    @pl.when(ti == 0)
    def _():
        h_ref[...] = jnp.zeros((batch, _H), jnp.float32)
        c_ref[...] = jnp.zeros((batch, _H), jnp.float32)

    half = (lax.broadcasted_iota(jnp.int32, (batch, d), 1)
            >= d // 2).astype(jnp.float32)
    h = h_ref[...]
    c = c_ref[...]
    for k in range(tt):
        m = p_ref[0, :, k:k + 1]
        mask = 1.0 - half - m + 2.0 * half * m
        xw = (x_ref[k] * mask).astype(jnp.bfloat16)
        z = jnp.dot(xw, w2_ref[...], preferred_element_type=jnp.float32)
        z = z + jnp.dot(h.astype(jnp.bfloat16), u_ref[...],
                        preferred_element_type=jnp.float32)
        z = z + b_ref[...]
        i = jax.nn.sigmoid(z[:, :_H])
        f = jax.nn.sigmoid(z[:, _H:2 * _H])
        g = jnp.tanh(z[:, 2 * _H:3 * _H])
        o = jax.nn.sigmoid(z[:, 3 * _H:])
        c = f * c + i * g
        h = o * jnp.tanh(c)
        o_ref[k, 0, :] = jax.nn.sigmoid(
            jnp.sum(h * wd_ref[...], axis=1) + bd_ref[0, 0])
    h_ref[...] = h
    c_ref[...] = c


def _lstm_fused(x2, parT, w2, u, b2, wdp, bdp, tt=8):
    t, batch, d = x2.shape
    grid = (t // tt,)
    par3 = jnp.transpose(parT.reshape(batch, t // tt, tt), (1, 0, 2))
    out = pl.pallas_call(
        functools.partial(_lstm_body, tt=tt, batch=batch, d=d),
        grid=grid,
        in_specs=[
            pl.BlockSpec((tt, batch, d), lambda ti: (ti, 0, 0)),
            pl.BlockSpec((1, batch, tt), lambda ti: (ti, 0, 0)),
            pl.BlockSpec(w2.shape, lambda ti: (0, 0)),
            pl.BlockSpec(u.shape, lambda ti: (0, 0)),
            pl.BlockSpec(b2.shape, lambda ti: (0, 0)),
            pl.BlockSpec(wdp.shape, lambda ti: (0, 0)),
            pl.BlockSpec(memory_space=pltpu.SMEM),
        ],
        out_specs=pl.BlockSpec((tt, 1, batch), lambda ti: (ti, 0, 0)),
        out_shape=jax.ShapeDtypeStruct((t, 1, batch), jnp.float32),
        scratch_shapes=[
            pltpu.VMEM((batch, _H), jnp.float32),
            pltpu.VMEM((batch, _H), jnp.float32),
        ],
        compiler_params=pltpu.CompilerParams(
            dimension_semantics=("arbitrary",)),
    )(x2, par3, w2, u, b2, wdp, bdp)
    return out


def kernel(reviews, emb, W, U, b, gamma, beta, moving_mean, moving_var,
           Wd, bd):
    batch, t = reviews.shape
    v, e = emb.shape
    n_full = v // (2 * e)
    tail_pairs = emb[n_full * 2 * e:].reshape(-1, 2 * e)
    pairs = _sc_transpose_pack(jnp.transpose(emb), tail_pairs)
    idxw = (jnp.transpose(reviews) >> 1).reshape(-1, _ROWS_PER_STREAM)
    parT = (reviews & 1).astype(jnp.float32)
    x2 = _sc_gather(pairs, idxw).reshape(t, batch, 2 * e)

    inv = gamma * lax.rsqrt(moving_var + 1e-3)
    wd0 = Wd[:, 0]
    wdp = (inv * wd0)[None, :]
    bdp = (bd[0] + jnp.sum((beta - inv * moving_mean) * wd0))[None, None]
    b2 = b[None, :]
    w2 = jnp.concatenate([W, W], axis=0).astype(jnp.bfloat16)

    out = _lstm_fused(x2, parT, w2, U.astype(jnp.bfloat16),
                      b2, wdp, bdp)
    return jnp.transpose(out.reshape(t, batch), (1, 0))[..., None]
